# baseline (speedup 1.0000x reference)
import functools

import jax
import jax.numpy as jnp
from jax import lax
from jax.experimental import pallas as pl
from jax.experimental.pallas import tpu as pltpu
from jax.experimental.pallas import tpu_sc as plsc

B = 4096
L = 50
DIM = 128
F = 100
VOCAB = 100000

_NC = 2
_NS = 16
_NW = _NC * _NS
_EPW = B // _NW

_mesh = plsc.VectorSubcoreMesh(core_axis_name="c", subcore_axis_name="s")


@functools.partial(
    pl.kernel,
    mesh=_mesh,
    out_type=jax.ShapeDtypeStruct((B, DIM), jnp.float32),
    scratch_types=[
        pltpu.VMEM((L, _EPW), jnp.int32),
        pltpu.VMEM((_EPW, DIM), jnp.float32),
        pltpu.SemaphoreType.DMA,
    ],
)
def _pool_sum(xT_hbm, embed_hbm, out_hbm, idx_v, acc_v, sem):
    wid = lax.axis_index("s") * _NC + lax.axis_index("c")
    base = wid * _EPW
    pltpu.sync_copy(xT_hbm.at[:, pl.ds(base, _EPW)], idx_v)
    pltpu.async_copy(embed_hbm.at[idx_v.at[0]], acc_v, sem).wait()
    copies = [
        pltpu.async_copy(embed_hbm.at[idx_v.at[l]], acc_v, sem, add=True)
        for l in range(1, L)
    ]
    for cp in copies:
        cp.wait()
    pltpu.sync_copy(acc_v, out_hbm.at[pl.ds(base, _EPW)])


def _h_body(msum_ref, w1_ref, b1_ref, out_ref):
    out_ref[...] = (
        jnp.dot(msum_ref[...], w1_ref[...], preferred_element_type=jnp.float32)
        * (1.0 / L)
        + b1_ref[...]
    )


def _out_body(h_ref, w2_ref, b2_ref, out_ref):
    out_ref[...] = (
        jnp.dot(h_ref[...], w2_ref[...], preferred_element_type=jnp.float32)
        + b2_ref[...]
    )


_BT = 1024
_VT = 1024


def kernel(x, embed, W1, b1, W2, b2):
    xT = jnp.transpose(x).astype(jnp.int32)
    msum = _pool_sum(xT, embed)

    h = pl.pallas_call(
        _h_body,
        out_shape=jax.ShapeDtypeStruct((B, F), jnp.float32),
    )(msum, W1, b1.reshape(1, F))

    nv = pl.cdiv(VOCAB, _VT)
    nb = B // _BT
    out = pl.pallas_call(
        _out_body,
        grid=(nv, nb),
        in_specs=[
            pl.BlockSpec((_BT, F), lambda v, b: (b, 0)),
            pl.BlockSpec((F, _VT), lambda v, b: (0, v)),
            pl.BlockSpec((1, _VT), lambda v, b: (0, v)),
        ],
        out_specs=pl.BlockSpec((_BT, _VT), lambda v, b: (b, v)),
        out_shape=jax.ShapeDtypeStruct((B, VOCAB), jnp.float32),
    )(h, W2, b2.reshape(1, VOCAB))
    return out


# speedup vs baseline: 1.0979x; 1.0979x over previous
import functools

import jax
import jax.numpy as jnp
from jax import lax
from jax.experimental import pallas as pl
from jax.experimental.pallas import tpu as pltpu
from jax.experimental.pallas import tpu_sc as plsc

B = 4096
L = 50
DIM = 128
F = 100
VOCAB = 100000

_NC = 2
_NS = 16
_NW = _NC * _NS
_EPW = B // _NW

_mesh = plsc.VectorSubcoreMesh(core_axis_name="c", subcore_axis_name="s")


@functools.partial(
    pl.kernel,
    mesh=_mesh,
    out_type=jax.ShapeDtypeStruct((B, DIM), jnp.float32),
    scratch_types=[
        pltpu.VMEM((L, _EPW), jnp.int32),
        pltpu.VMEM((_EPW, DIM), jnp.float32),
        pltpu.SemaphoreType.DMA,
    ],
)
def _pool_sum(xT_hbm, embed_hbm, out_hbm, idx_v, acc_v, sem):
    wid = lax.axis_index("s") * _NC + lax.axis_index("c")
    base = wid * _EPW
    pltpu.sync_copy(xT_hbm.at[:, pl.ds(base, _EPW)], idx_v)
    pltpu.async_copy(embed_hbm.at[idx_v.at[0]], acc_v, sem).wait()
    copies = [
        pltpu.async_copy(embed_hbm.at[idx_v.at[l]], acc_v, sem, add=True)
        for l in range(1, L)
    ]
    for cp in copies:
        cp.wait()
    pltpu.sync_copy(acc_v, out_hbm.at[pl.ds(base, _EPW)])


def _h_body(msum_ref, w1_ref, b1_ref, out_ref):
    out_ref[...] = (
        jnp.dot(msum_ref[...], w1_ref[...], preferred_element_type=jnp.float32)
        * (1.0 / L)
        + b1_ref[...]
    )


def _out_body(h_ref, w2_ref, b2_ref, out_ref):
    out_ref[...] = (
        jnp.dot(h_ref[...], w2_ref[...], preferred_element_type=jnp.float32)
        + b2_ref[...]
    )


_BT = 4096
_VT = 1024


def kernel(x, embed, W1, b1, W2, b2):
    xT = jnp.transpose(x).astype(jnp.int32)
    msum = _pool_sum(xT, embed)

    h = pl.pallas_call(
        _h_body,
        out_shape=jax.ShapeDtypeStruct((B, F), jnp.float32),
    )(msum, W1, b1.reshape(1, F))

    nv = pl.cdiv(VOCAB, _VT)
    out = pl.pallas_call(
        _out_body,
        grid=(nv,),
        in_specs=[
            pl.BlockSpec((_BT, F), lambda v: (0, 0)),
            pl.BlockSpec((F, _VT), lambda v: (0, v)),
            pl.BlockSpec((1, _VT), lambda v: (0, v)),
        ],
        out_specs=pl.BlockSpec((_BT, _VT), lambda v: (0, v)),
        out_shape=jax.ShapeDtypeStruct((B, VOCAB), jnp.float32),
    )(h, W2, b2.reshape(1, VOCAB))
    return out
